# Initial kernel scaffold; baseline (speedup 1.0000x reference)
import jax
import jax.numpy as jnp
from jax.experimental import pallas as pl


def kernel(g, x, w_self0, w_neigh0, b0, w_self1, w_neigh1, b1, w_self2, w_neigh2, b2):
    raise NotImplementedError("write your pallas kernel here")


# speedup vs baseline: 6.6725x; 6.6725x over previous
import functools

import jax
import jax.numpy as jnp
from jax import lax
from jax.experimental import pallas as pl
from jax.experimental.pallas import tpu as pltpu
from jax.experimental.pallas import tpu_sc as plsc

N_NODES = 10000
D = 128
N_PAD = 10240
ROWS_PER_TILE = N_PAD // 16
E = 320000
NW = 32
E_PER_W = E // NW
CHUNK = 128
N_CHUNKS = E_PER_W // CHUNK
TAIL = E_PER_W - N_CHUNKS * CHUNK

f32 = jnp.float32


def _make_segsum(with_deg):
    out_types = [jax.ShapeDtypeStruct((2 * N_PAD, D), f32)]
    scratch = [
        pltpu.VMEM_SHARED((N_PAD, D), f32),
        pltpu.VMEM((CHUNK,), jnp.int32),
        pltpu.VMEM((CHUNK,), jnp.int32),
        pltpu.VMEM((CHUNK, D), f32),
        pltpu.VMEM((TAIL,), jnp.int32),
        pltpu.VMEM((TAIL,), jnp.int32),
        pltpu.VMEM((TAIL, D), f32),
        pltpu.VMEM((CHUNK, D), f32),
        pltpu.SemaphoreType.DMA,
    ]
    if with_deg:
        out_types.append(jax.ShapeDtypeStruct((2 * N_PAD,), f32))
        scratch += [
            pltpu.VMEM_SHARED((N_PAD,), f32),
            pltpu.VMEM((CHUNK,), f32),
            pltpu.VMEM((TAIL,), f32),
            pltpu.VMEM((ROWS_PER_TILE,), f32),
        ]
    mesh = plsc.VectorSubcoreMesh(core_axis_name="c", subcore_axis_name="s")

    def body(x_hbm, src_hbm, dst_hbm, *rest):
        if with_deg:
            (parts_out, deg_out, acc_sh, src_v, dst_v, rows_v, src_t, dst_t,
             rows_t, zrow_v, gsem, deg_sh, ones_v, ones_t, zvec_v) = rest
        else:
            (parts_out, acc_sh, src_v, dst_v, rows_v, src_t, dst_t,
             rows_t, zrow_v, gsem) = rest

        c = lax.axis_index("c")
        s = lax.axis_index("s")
        wid = c * 16 + s
        row0 = s * ROWS_PER_TILE

        def zrow_loop(i, carry):
            for j in range(D // 16):
                zrow_v[i, pl.ds(j * 16, 16)] = jnp.zeros((16,), f32)
            return carry
        lax.fori_loop(0, CHUNK, zrow_loop, 0)
        for j in range(ROWS_PER_TILE // CHUNK):
            pltpu.sync_copy(zrow_v, acc_sh.at[pl.ds(row0 + j * CHUNK, CHUNK)])

        if with_deg:
            for j in range(ROWS_PER_TILE // 16):
                zvec_v[pl.ds(j * 16, 16)] = jnp.zeros((16,), f32)
            pltpu.sync_copy(zvec_v, deg_sh.at[pl.ds(row0, ROWS_PER_TILE)])
            for j in range(CHUNK // 16):
                ones_v[pl.ds(j * 16, 16)] = jnp.ones((16,), f32)
            ones_t[pl.ds(0, TAIL)] = jnp.ones((TAIL,), f32)

        plsc.subcore_barrier()

        e0 = wid * E_PER_W

        def chunk_body(i, carry):
            base = e0 + i * CHUNK
            pltpu.sync_copy(src_hbm.at[pl.ds(base, CHUNK)], src_v)
            pltpu.sync_copy(dst_hbm.at[pl.ds(base, CHUNK)], dst_v)
            pltpu.async_copy(x_hbm.at[src_v], rows_v, gsem).wait()
            pltpu.sync_copy(rows_v, acc_sh.at[dst_v], add=True)
            if with_deg:
                pltpu.sync_copy(ones_v, deg_sh.at[dst_v], add=True)
            return carry
        lax.fori_loop(0, N_CHUNKS, chunk_body, 0)

        base = e0 + N_CHUNKS * CHUNK
        pltpu.sync_copy(src_hbm.at[pl.ds(base, TAIL)], src_t)
        pltpu.sync_copy(dst_hbm.at[pl.ds(base, TAIL)], dst_t)
        pltpu.async_copy(x_hbm.at[src_t], rows_t, gsem).wait()
        pltpu.sync_copy(rows_t, acc_sh.at[dst_t], add=True)
        if with_deg:
            pltpu.sync_copy(ones_t, deg_sh.at[dst_t], add=True)

        plsc.subcore_barrier()

        pltpu.sync_copy(acc_sh.at[pl.ds(row0, ROWS_PER_TILE)],
                        parts_out.at[pl.ds(c * N_PAD + row0, ROWS_PER_TILE)])
        if with_deg:
            pltpu.sync_copy(deg_sh.at[pl.ds(row0, ROWS_PER_TILE)],
                            deg_out.at[pl.ds(c * N_PAD + row0, ROWS_PER_TILE)])

    return pl.kernel(body, out_type=out_types, mesh=mesh,
                     scratch_types=scratch)


RB = 2560


def _make_dense(relu):
    def body(x_ref, p_ref, dcol_ref, ws_ref, wn_ref, b_ref, o_ref):
        deg = jnp.maximum(dcol_ref[...], 1.0)
        agg = (p_ref[0] + p_ref[1]) / deg
        h = (jnp.dot(x_ref[...], ws_ref[...], preferred_element_type=f32)
             + jnp.dot(agg, wn_ref[...], preferred_element_type=f32)
             + b_ref[...])
        o_ref[...] = jnp.maximum(h, 0.0) if relu else h

    return pl.pallas_call(
        body,
        grid=(N_PAD // RB,),
        in_specs=[
            pl.BlockSpec((RB, D), lambda i: (i, 0)),
            pl.BlockSpec((2, RB, D), lambda i: (0, i, 0)),
            pl.BlockSpec((RB, 1), lambda i: (i, 0)),
            pl.BlockSpec((D, D), lambda i: (0, 0)),
            pl.BlockSpec((D, D), lambda i: (0, 0)),
            pl.BlockSpec((1, D), lambda i: (0, 0)),
        ],
        out_specs=pl.BlockSpec((RB, D), lambda i: (i, 0)),
        out_shape=jax.ShapeDtypeStruct((N_PAD, D), f32),
    )


_segsum_deg = _make_segsum(True)
_segsum = _make_segsum(False)
_dense_relu = _make_dense(True)
_dense_lin = _make_dense(False)


def kernel(g, x, w_self0, w_neigh0, b0, w_self1, w_neigh1, b1,
           w_self2, w_neigh2, b2):
    src = g[0].astype(jnp.int32)
    dst = g[1].astype(jnp.int32)
    xp = jnp.pad(x, ((0, N_PAD - N_NODES), (0, 0)))

    parts, deg = _segsum_deg(xp, src, dst)
    dcol = (deg[:N_PAD] + deg[N_PAD:]).reshape(N_PAD, 1)

    h = _dense_relu(xp, parts.reshape(2, N_PAD, D), dcol,
                    w_self0, w_neigh0, b0.reshape(1, D))
    parts, = _segsum(h, src, dst)
    h = _dense_relu(h, parts.reshape(2, N_PAD, D), dcol,
                    w_self1, w_neigh1, b1.reshape(1, D))
    parts, = _segsum(h, src, dst)
    h = _dense_lin(h, parts.reshape(2, N_PAD, D), dcol,
                   w_self2, w_neigh2, b2.reshape(1, D))
    return h[:N_NODES]


# speedup vs baseline: 14.2605x; 2.1372x over previous
import functools

import jax
import jax.numpy as jnp
from jax import lax
from jax.experimental import pallas as pl
from jax.experimental.pallas import tpu as pltpu
from jax.experimental.pallas import tpu_sc as plsc

N_NODES = 10000
D = 128
N_PAD = 10240
ROWS_PER_TILE = N_PAD // 16
E = 320000
NW = 32
CHUNK = 64
CPW = 160
PCH = 40
E_PAD = NW * CPW * CHUNK
NBUF = 4
GROUPS = PCH // NBUF

f32 = jnp.float32


def _make_segsum(with_deg):
    out_types = [jax.ShapeDtypeStruct((2 * N_PAD, D), f32)]
    scratch = [
        pltpu.VMEM_SHARED((N_PAD, D), f32),
        pltpu.VMEM((PCH, CHUNK), jnp.int32),
        pltpu.VMEM((PCH, CHUNK), jnp.int32),
        [pltpu.VMEM((CHUNK, D), f32) for _ in range(NBUF)],
        [pltpu.SemaphoreType.DMA for _ in range(NBUF)],
        pltpu.SemaphoreType.DMA,
        pltpu.SemaphoreType.DMA,
    ]
    if with_deg:
        out_types.append(jax.ShapeDtypeStruct((2 * N_PAD,), f32))
        scratch += [
            pltpu.VMEM_SHARED((N_PAD,), f32),
            pltpu.VMEM((CHUNK,), f32),
            pltpu.VMEM((ROWS_PER_TILE,), f32),
        ]
    mesh = plsc.VectorSubcoreMesh(core_axis_name="c", subcore_axis_name="s")

    def body(x_hbm, src_hbm, dst_hbm, *rest):
        if with_deg:
            (parts_out, deg_out, acc_sh, src2d, dst2d, rowsb,
             gsem, ssem, isem, deg_sh, ones_v, zvec_v) = rest
        else:
            (parts_out, acc_sh, src2d, dst2d, rowsb,
             gsem, ssem, isem) = rest

        c = lax.axis_index("c")
        s = lax.axis_index("s")
        wid = c * 16 + s
        row0 = s * ROWS_PER_TILE
        crow0 = wid * CPW

        pltpu.async_copy(src_hbm.at[pl.ds(crow0, PCH)], src2d, isem)
        pltpu.async_copy(dst_hbm.at[pl.ds(crow0, PCH)], dst2d, isem)

        def zrow_loop(i, carry):
            for j in range(D // 16):
                rowsb[0][i, pl.ds(j * 16, 16)] = jnp.zeros((16,), f32)
            return carry
        lax.fori_loop(0, CHUNK, zrow_loop, 0)
        for j in range(ROWS_PER_TILE // CHUNK):
            pltpu.sync_copy(rowsb[0],
                            acc_sh.at[pl.ds(row0 + j * CHUNK, CHUNK)])

        if with_deg:
            for j in range(ROWS_PER_TILE // 16):
                zvec_v[pl.ds(j * 16, 16)] = jnp.zeros((16,), f32)
            pltpu.sync_copy(zvec_v, deg_sh.at[pl.ds(row0, ROWS_PER_TILE)])
            for j in range(CHUNK // 16):
                ones_v[pl.ds(j * 16, 16)] = jnp.ones((16,), f32)

        pltpu.make_async_copy(src_hbm.at[pl.ds(crow0, PCH)], src2d,
                              isem).wait()
        pltpu.make_async_copy(dst_hbm.at[pl.ds(crow0, PCH)], dst2d,
                              isem).wait()
        plsc.subcore_barrier()

        def g_start(ci, p):
            pltpu.async_copy(x_hbm.at[src2d.at[ci]], rowsb[p], gsem[p])

        def g_wait(ci, p):
            pltpu.make_async_copy(x_hbm.at[src2d.at[ci]], rowsb[p],
                                  gsem[p]).wait()

        def s_start(ci, p):
            pltpu.async_copy(rowsb[p], acc_sh.at[dst2d.at[ci]], ssem,
                             add=True)

        def s_wait(ci, p):
            pltpu.make_async_copy(rowsb[p], acc_sh.at[dst2d.at[ci]],
                                  ssem).wait()

        def step(ci, b, do_swait, do_gstart):
            q = (b + NBUF - 1) % NBUF
            if do_swait:
                s_wait(ci - 1, q)
            if do_gstart:
                g_start(ci + NBUF - 1, q)
            g_wait(ci, b)
            s_start(ci, b)
            if with_deg:
                pltpu.sync_copy(ones_v, deg_sh.at[dst2d.at[ci]], add=True)

        for phase in range(CPW // PCH):
            if phase > 0:
                pltpu.async_copy(src_hbm.at[pl.ds(crow0 + phase * PCH, PCH)],
                                 src2d, isem)
                pltpu.async_copy(dst_hbm.at[pl.ds(crow0 + phase * PCH, PCH)],
                                 dst2d, isem)
                pltpu.make_async_copy(src_hbm.at[pl.ds(crow0, PCH)], src2d,
                                      isem).wait()
                pltpu.make_async_copy(dst_hbm.at[pl.ds(crow0, PCH)], dst2d,
                                      isem).wait()

            for b in range(NBUF - 1):
                g_start(b, b)

            for b in range(NBUF):
                step(b, b, do_swait=(b >= 1), do_gstart=True)

            def group_body(g, carry):
                i0 = g * NBUF
                for b in range(NBUF):
                    step(i0 + b, b, do_swait=True, do_gstart=True)
                return carry
            lax.fori_loop(1, GROUPS - 1, group_body, 0)

            i0 = (GROUPS - 1) * NBUF
            for b in range(NBUF):
                step(i0 + b, b, do_swait=True, do_gstart=(b == 0))
            s_wait(PCH - 1, (PCH - 1) % NBUF)

        plsc.subcore_barrier()

        pltpu.sync_copy(acc_sh.at[pl.ds(row0, ROWS_PER_TILE)],
                        parts_out.at[pl.ds(c * N_PAD + row0, ROWS_PER_TILE)])
        if with_deg:
            pltpu.sync_copy(deg_sh.at[pl.ds(row0, ROWS_PER_TILE)],
                            deg_out.at[pl.ds(c * N_PAD + row0, ROWS_PER_TILE)])

    return pl.kernel(body, out_type=out_types, mesh=mesh,
                     scratch_types=scratch)


RB = 2560


def _make_dense(relu):
    def body(x_ref, p_ref, dcol_ref, ws_ref, wn_ref, b_ref, o_ref):
        deg = jnp.maximum(dcol_ref[...], 1.0)
        agg = (p_ref[0] + p_ref[1]) / deg
        h = (jnp.dot(x_ref[...], ws_ref[...], preferred_element_type=f32)
             + jnp.dot(agg, wn_ref[...], preferred_element_type=f32)
             + b_ref[...])
        o_ref[...] = jnp.maximum(h, 0.0) if relu else h

    return pl.pallas_call(
        body,
        grid=(N_PAD // RB,),
        in_specs=[
            pl.BlockSpec((RB, D), lambda i: (i, 0)),
            pl.BlockSpec((2, RB, D), lambda i: (0, i, 0)),
            pl.BlockSpec((RB, 1), lambda i: (i, 0)),
            pl.BlockSpec((D, D), lambda i: (0, 0)),
            pl.BlockSpec((D, D), lambda i: (0, 0)),
            pl.BlockSpec((1, D), lambda i: (0, 0)),
        ],
        out_specs=pl.BlockSpec((RB, D), lambda i: (i, 0)),
        out_shape=jax.ShapeDtypeStruct((N_PAD, D), f32),
    )


_segsum_deg = _make_segsum(True)
_segsum = _make_segsum(False)
_dense_relu = _make_dense(True)
_dense_lin = _make_dense(False)


def kernel(g, x, w_self0, w_neigh0, b0, w_self1, w_neigh1, b1,
           w_self2, w_neigh2, b2):
    src = g[0].astype(jnp.int32)
    dst = g[1].astype(jnp.int32)
    pad_n = E_PAD - E
    pad_ids = jnp.arange(pad_n, dtype=jnp.int32)
    src_r = jnp.concatenate([src, pad_ids % N_NODES]).reshape(E_PAD // CHUNK,
                                                              CHUNK)
    dst_r = jnp.concatenate(
        [dst, N_NODES + pad_ids % (N_PAD - N_NODES)]).reshape(E_PAD // CHUNK,
                                                              CHUNK)
    xp = jnp.pad(x, ((0, N_PAD - N_NODES), (0, 0)))

    parts, deg = _segsum_deg(xp, src_r, dst_r)
    dcol = (deg[:N_PAD] + deg[N_PAD:]).reshape(N_PAD, 1)

    h = _dense_relu(xp, parts.reshape(2, N_PAD, D), dcol,
                    w_self0, w_neigh0, b0.reshape(1, D))
    parts, = _segsum(h, src_r, dst_r)
    h = _dense_relu(h, parts.reshape(2, N_PAD, D), dcol,
                    w_self1, w_neigh1, b1.reshape(1, D))
    parts, = _segsum(h, src_r, dst_r)
    h = _dense_lin(h, parts.reshape(2, N_PAD, D), dcol,
                   w_self2, w_neigh2, b2.reshape(1, D))
    return h[:N_NODES]
